# Initial kernel scaffold; baseline (speedup 1.0000x reference)
import jax
import jax.numpy as jnp
from jax.experimental import pallas as pl


def kernel(x, table):
    raise NotImplementedError("write your pallas kernel here")


# speedup vs baseline: 5.0996x; 5.0996x over previous
import functools
import math

import jax
import jax.numpy as jnp
from jax import lax
from jax.experimental import pallas as pl
from jax.experimental.pallas import tpu as pltpu
from jax.experimental.pallas import tpu_sc as plsc

VOCAB = 100000
D = 128
B_TOTAL = 4096 * 200
NC, NS = 2, 16
NW = NC * NS
B_PER_W = B_TOTAL // NW
CHUNK = 128
NCHUNK = B_PER_W // CHUNK
SCALE = math.sqrt(float(D))
LANES = 16


def _embed_body(x_hbm, table_hbm, out_hbm, idx_v, rows_v, gsem, osem):
    wid = lax.axis_index("s") * NC + lax.axis_index("c")

    pltpu.sync_copy(x_hbm.at[wid], idx_v)

    def chunk_body(j, _):
        pltpu.async_copy(table_hbm.at[idx_v.at[j]], rows_v, gsem).wait()

        def scale_row(r, _):
            for c in range(D // LANES):
                sl = pl.ds(c * LANES, LANES)
                rows_v[r, sl] = rows_v[r, sl] * SCALE
            return 0

        lax.fori_loop(0, CHUNK, scale_row, 0)
        pltpu.sync_copy(rows_v, out_hbm.at[wid, j])
        return 0

    lax.fori_loop(0, NCHUNK, chunk_body, 0)


@functools.partial(jax.jit, donate_argnums=())
def kernel(x, table):
    x3 = x.astype(jnp.int32).reshape(NW, NCHUNK, CHUNK)
    grid_kernel = pl.kernel(
        _embed_body,
        out_type=jax.ShapeDtypeStruct((NW, NCHUNK, CHUNK, D), jnp.float32),
        mesh=plsc.VectorSubcoreMesh(
            core_axis_name="c", subcore_axis_name="s",
            num_cores=NC, num_subcores=NS,
        ),
        scratch_types=[
            pltpu.VMEM((NCHUNK, CHUNK), jnp.int32),
            pltpu.VMEM((CHUNK, D), jnp.float32),
            pltpu.SemaphoreType.DMA,
            pltpu.SemaphoreType.DMA,
        ],
    )
    out = grid_kernel(x3, table)
    return out.reshape(4096, 200, D)


# speedup vs baseline: 9.2098x; 1.8060x over previous
import functools
import math

import jax
import jax.numpy as jnp
from jax import lax
from jax.experimental import pallas as pl
from jax.experimental.pallas import tpu as pltpu
from jax.experimental.pallas import tpu_sc as plsc

VOCAB = 100000
D = 128
B_TOTAL = 4096 * 200
NC, NS = 2, 16
NW = NC * NS
B_PER_W = B_TOTAL // NW
CHUNK = 128
NCHUNK = B_PER_W // CHUNK
SCALE = math.sqrt(float(D))
LANES = 16


def _embed_body(x_hbm, table_hbm, out_hbm, idx_v,
                in0, in1, out0, out1, gsem0, gsem1, osem0, osem1):
    wid = lax.axis_index("s") * NC + lax.axis_index("c")
    inb, outb = (in0, in1), (out0, out1)
    gsems, osems = (gsem0, gsem1), (osem0, osem1)

    pltpu.sync_copy(x_hbm.at[wid], idx_v)

    def gather(i, b):
        return pltpu.async_copy(table_hbm.at[idx_v.at[i]], inb[b], gsems[b])

    def ocopy(i, b):
        return pltpu.make_async_copy(outb[b], out_hbm.at[wid, i], osems[b])

    def step(i, b, wait_out, issue_next):
        pltpu.make_async_copy(table_hbm.at[idx_v.at[i]], inb[b], gsems[b]).wait()
        if wait_out:
            ocopy(i, b).wait()

        def scale_row(r, _):
            for c in range(D // LANES):
                sl = pl.ds(c * LANES, LANES)
                outb[b][r, sl] = inb[b][r, sl] * SCALE
            return 0

        lax.fori_loop(0, CHUNK, scale_row, 0)
        ocopy(i, b).start()
        if issue_next:
            gather(i + 2, b)

    gather(0, 0)
    gather(1, 1)
    step(0, 0, False, True)
    step(1, 1, False, True)

    def loop_body(t, _):
        step(2 * t, 0, True, True)
        step(2 * t + 1, 1, True, True)
        return 0

    lax.fori_loop(1, NCHUNK // 2 - 1, loop_body, 0)
    step(NCHUNK - 2, 0, True, False)
    step(NCHUNK - 1, 1, True, False)
    ocopy(NCHUNK - 2, 0).wait()
    ocopy(NCHUNK - 1, 1).wait()


@functools.partial(jax.jit, donate_argnums=())
def kernel(x, table):
    x3 = x.astype(jnp.int32).reshape(NW, NCHUNK, CHUNK)
    grid_kernel = pl.kernel(
        _embed_body,
        out_type=jax.ShapeDtypeStruct((NW, NCHUNK, CHUNK, D), jnp.float32),
        mesh=plsc.VectorSubcoreMesh(
            core_axis_name="c", subcore_axis_name="s",
            num_cores=NC, num_subcores=NS,
        ),
        scratch_types=[
            pltpu.VMEM((NCHUNK, CHUNK), jnp.int32),
            pltpu.VMEM((CHUNK, D), jnp.float32),
            pltpu.VMEM((CHUNK, D), jnp.float32),
            pltpu.VMEM((CHUNK, D), jnp.float32),
            pltpu.VMEM((CHUNK, D), jnp.float32),
            pltpu.SemaphoreType.DMA,
            pltpu.SemaphoreType.DMA,
            pltpu.SemaphoreType.DMA,
            pltpu.SemaphoreType.DMA,
        ],
    )
    out = grid_kernel(x3, table)
    return out.reshape(4096, 200, D)
